# baseline (speedup 1.0000x reference)
import dataclasses
import functools

import jax
import jax.numpy as jnp
from jax import lax
from jax.experimental import pallas as pl
from jax.experimental.pallas import tpu as pltpu
from jax.experimental.pallas import tpu_sc as plsc

_N_L1 = 512
_N_PSQT = 8
_TOTAL = _N_L1 + _N_PSQT
_ROWS = 100000
_BATCH = 4096
_ACTIVE = 20
_NC = 2
_NS = 16
_NW = _NC * _NS
_LANES = 16
_SB = 512
_NSB = _BATCH // _SB
_MAGIC = 12582912.0
_MAGIC20 = _MAGIC * _ACTIVE
_NCOL_HI = 17
_NCOL_LO = 16
_HI_WORKERS = _TOTAL - _NCOL_LO * _NW


def _sc_cols(idxT, wT, rinv, sfin, bias):
    mesh = plsc.VectorSubcoreMesh(core_axis_name="c", subcore_axis_name="s")
    cp = pltpu.CompilerParams()
    if "needs_layout_passes" in pltpu.CompilerParams.__dataclass_fields__:
        cp = dataclasses.replace(cp, needs_layout_passes=False)

    @functools.partial(
        pl.kernel,
        out_type=jax.ShapeDtypeStruct((_TOTAL, _BATCH), jnp.float32),
        mesh=mesh,
        compiler_params=cp,
        scratch_types=[
            pltpu.VMEM((_ROWS,), jnp.float32),
            pltpu.VMEM((_ACTIVE, _SB), jnp.int32),
            pltpu.VMEM((_ACTIVE, _SB), jnp.int32),
            pltpu.VMEM((_BATCH,), jnp.float32),
            pltpu.VMEM((_TOTAL,), jnp.float32),
            pltpu.VMEM((_TOTAL,), jnp.float32),
            pltpu.VMEM((_TOTAL,), jnp.float32),
            pltpu.SemaphoreType.DMA,
            pltpu.SemaphoreType.DMA,
            pltpu.SemaphoreType.DMA,
            pltpu.SemaphoreType.DMA,
            pltpu.SemaphoreType.DMA,
            pltpu.SemaphoreType.DMA,
        ],
    )
    def body(idxT_hbm, wT_hbm, rinv_hbm, sfin_hbm, bias_hbm, outT_hbm,
             colbuf, idx0, idx1, outbuf, rinv_s, sfin_s, bias_s,
             csem0, csem1, csem2, csem3, isem0, isem1):
        wid = lax.axis_index("s") * _NC + lax.axis_index("c")
        nt = jnp.where(wid < _HI_WORKERS, _NCOL_HI, _NCOL_LO)
        start = jnp.where(wid < _HI_WORKERS,
                          _NCOL_HI * wid,
                          _NCOL_LO * wid + _HI_WORKERS)
        pltpu.sync_copy(rinv_hbm, rinv_s)
        pltpu.sync_copy(sfin_hbm, sfin_s)
        pltpu.sync_copy(bias_hbm, bias_s)

        def idx_start(sb, buf, sem):
            pltpu.async_copy(idxT_hbm.at[:, pl.ds(sb * _SB, _SB)], buf, sem)

        def idx_wait(buf, sem):
            pltpu.make_async_copy(
                idxT_hbm.at[:, pl.ds(0, _SB)], buf, sem).wait()

        @pl.loop(0, _NCOL_HI)
        def _(k):
            @pl.when(k < nt)
            def _():
                c = start + k
                pltpu.async_copy(wT_hbm.at[c], colbuf, csem0)
                idx_start(0, idx0, isem0)
                cvec = jnp.full((_LANES,), c, jnp.int32)
                rv = plsc.load_gather(rinv_s, [cvec])
                sv = plsc.load_gather(sfin_s, [cvec])
                bv = plsc.load_gather(bias_s, [cvec])
                mvec = jnp.full((_LANES,), _MAGIC, jnp.float32)
                m20v = jnp.full((_LANES,), _MAGIC20, jnp.float32)
                pltpu.make_async_copy(wT_hbm.at[c], colbuf, csem0).wait()

                def do_block(sb, buf):
                    @plsc.parallel_loop(0, _SB // _LANES, unroll=2)
                    def _(s16):
                        accs = [jnp.zeros((_LANES,), jnp.float32)
                                for _ in range(4)]
                        for a in range(_ACTIVE):
                            iv = buf[a, pl.ds(s16 * _LANES, _LANES)]
                            vals = plsc.load_gather(colbuf, [iv])
                            accs[a % 4] = accs[a % 4] + (vals * rv + mvec)
                        acc = (accs[0] + accs[1]) + (accs[2] + accs[3])
                        outbuf[pl.ds(sb * _SB + s16 * _LANES, _LANES)] = (
                            (acc - m20v) * sv + bv)

                for sb in range(_NSB):
                    cur, nxt = (idx0, idx1) if sb % 2 == 0 else (idx1, idx0)
                    csem_cur, csem_nxt = (
                        (isem0, isem1) if sb % 2 == 0 else (isem1, isem0))
                    idx_wait(cur, csem_cur)
                    if sb + 1 < _NSB:
                        idx_start(sb + 1, nxt, csem_nxt)
                    do_block(sb, cur)

                pltpu.sync_copy(outbuf, outT_hbm.at[c])

    return body(idxT, wT, rinv, sfin, bias)


def kernel(indices, weight, bias, scale_l1, scale_psqt):
    s_full = jnp.concatenate([scale_l1, scale_psqt]).astype(jnp.float32)
    rinv_full = (1.0 / s_full).astype(jnp.float32)
    wT = weight.T
    idxT = indices.T
    outT = _sc_cols(idxT, wT, rinv_full, s_full, bias.astype(jnp.float32))
    return outT.T


# speedup vs baseline: 1.0121x; 1.0121x over previous
import dataclasses
import functools

import jax
import jax.numpy as jnp
from jax import lax
from jax.experimental import pallas as pl
from jax.experimental.pallas import tpu as pltpu
from jax.experimental.pallas import tpu_sc as plsc

_N_L1 = 512
_N_PSQT = 8
_TOTAL = _N_L1 + _N_PSQT
_ROWS = 100000
_BATCH = 4096
_ACTIVE = 20
_NC = 2
_NS = 16
_NW = _NC * _NS
_LANES = 16
_SB = 512
_NSB = _BATCH // _SB
_MAGIC = 12582912.0
_MAGIC20 = _MAGIC * _ACTIVE
_NCOL_HI = 17
_NCOL_LO = 16
_HI_WORKERS = _TOTAL - _NCOL_LO * _NW


def _sc_cols(idxT, wT, rinv, sfin, bias):
    mesh = plsc.VectorSubcoreMesh(core_axis_name="c", subcore_axis_name="s")
    cp = pltpu.CompilerParams()
    if "needs_layout_passes" in pltpu.CompilerParams.__dataclass_fields__:
        cp = dataclasses.replace(cp, needs_layout_passes=False)

    @functools.partial(
        pl.kernel,
        out_type=jax.ShapeDtypeStruct((_TOTAL, _BATCH), jnp.float32),
        mesh=mesh,
        compiler_params=cp,
        scratch_types=[
            pltpu.VMEM((_ROWS,), jnp.float32),
            pltpu.VMEM((_ACTIVE, _SB), jnp.int32),
            pltpu.VMEM((_ACTIVE, _SB), jnp.int32),
            pltpu.VMEM((_BATCH,), jnp.float32),
            pltpu.VMEM((_TOTAL,), jnp.float32),
            pltpu.VMEM((_TOTAL,), jnp.float32),
            pltpu.VMEM((_TOTAL,), jnp.float32),
            pltpu.SemaphoreType.DMA,
            pltpu.SemaphoreType.DMA,
            pltpu.SemaphoreType.DMA,
            pltpu.SemaphoreType.DMA,
            pltpu.SemaphoreType.DMA,
            pltpu.SemaphoreType.DMA,
        ],
    )
    def body(idxT_hbm, wT_hbm, rinv_hbm, sfin_hbm, bias_hbm, outT_hbm,
             colbuf, idx0, idx1, outbuf, rinv_s, sfin_s, bias_s,
             csem0, csem1, csem2, csem3, isem0, isem1):
        wid = lax.axis_index("s") * _NC + lax.axis_index("c")
        nt = jnp.where(wid < _HI_WORKERS, _NCOL_HI, _NCOL_LO)
        start = jnp.where(wid < _HI_WORKERS,
                          _NCOL_HI * wid,
                          _NCOL_LO * wid + _HI_WORKERS)
        pltpu.sync_copy(rinv_hbm, rinv_s)
        pltpu.sync_copy(sfin_hbm, sfin_s)
        pltpu.sync_copy(bias_hbm, bias_s)

        def idx_start(sb, buf, sem):
            pltpu.async_copy(idxT_hbm.at[:, pl.ds(sb * _SB, _SB)], buf, sem)

        def idx_wait(buf, sem):
            pltpu.make_async_copy(
                idxT_hbm.at[:, pl.ds(0, _SB)], buf, sem).wait()

        @pl.loop(0, _NCOL_HI)
        def _(k):
            @pl.when(k < nt)
            def _():
                c = start + k
                pltpu.async_copy(wT_hbm.at[c], colbuf, csem0)
                idx_start(0, idx0, isem0)
                cvec = jnp.full((_LANES,), c, jnp.int32)
                rv = plsc.load_gather(rinv_s, [cvec])
                sv = plsc.load_gather(sfin_s, [cvec])
                bv = plsc.load_gather(bias_s, [cvec])
                mvec = jnp.full((_LANES,), _MAGIC, jnp.float32)
                m20v = jnp.full((_LANES,), _MAGIC20, jnp.float32)
                pltpu.make_async_copy(wT_hbm.at[c], colbuf, csem0).wait()

                def do_block(sb, buf):
                    @plsc.parallel_loop(0, _SB // _LANES)
                    def _(s16):
                        accs = [jnp.zeros((_LANES,), jnp.float32)
                                for _ in range(4)]
                        for a in range(_ACTIVE):
                            iv = buf[a, pl.ds(s16 * _LANES, _LANES)]
                            vals = plsc.load_gather(colbuf, [iv])
                            accs[a % 4] = accs[a % 4] + (vals * rv + mvec)
                        acc = (accs[0] + accs[1]) + (accs[2] + accs[3])
                        outbuf[pl.ds(sb * _SB + s16 * _LANES, _LANES)] = (
                            (acc - m20v) * sv + bv)

                for sb in range(_NSB):
                    cur, nxt = (idx0, idx1) if sb % 2 == 0 else (idx1, idx0)
                    csem_cur, csem_nxt = (
                        (isem0, isem1) if sb % 2 == 0 else (isem1, isem0))
                    idx_wait(cur, csem_cur)
                    if sb + 1 < _NSB:
                        idx_start(sb + 1, nxt, csem_nxt)
                    do_block(sb, cur)

                pltpu.sync_copy(outbuf, outT_hbm.at[c])

    return body(idxT, wT, rinv, sfin, bias)


def kernel(indices, weight, bias, scale_l1, scale_psqt):
    s_full = jnp.concatenate([scale_l1, scale_psqt]).astype(jnp.float32)
    rinv_full = (1.0 / s_full).astype(jnp.float32)
    wT = weight.T
    idxT = indices.T
    outT = _sc_cols(idxT, wT, rinv_full, s_full, bias.astype(jnp.float32))
    return outT.T
